# baseline (speedup 1.0000x reference)
import jax
import jax.numpy as jnp
import numpy as np
from jax.experimental import pallas as pl
from jax.experimental.pallas import tpu as pltpu

N_SA = 512; N_RA = 512; N_IN = 128; N_PN = 128; N_CN = 64
T = 120; MAX_DELAY = 10; BUF = MAX_DELAY + 1
TS = T - 1
PAD = 16
NBLK = (TS + 7) // 8
HROWS = PAD + 8 * NBLK + 8
CROWS = 8 * NBLK
V_THRES = 30.0

_HI = jax.lax.Precision.HIGHEST
_DN_RR = (((1,), (1,)), ((), ()))


def _decay_filter(tau: float) -> np.ndarray:
    e = np.float32(np.exp(np.float32(-1.0) / np.float32(tau)))
    k = np.arange(TS)
    pw = np.power(np.float64(e), k).astype(np.float32)
    L = np.zeros((TS, TS), np.float32)
    for r in range(TS):
        L[r, : r + 1] = pw[: r + 1][::-1]
    return L


def _a_schedule() -> np.ndarray:
    a = np.float32(0.02)
    out = np.zeros((CROWS, 1), np.float32)
    for r in range(TS):
        out[r, 0] = a
        a = np.float32(a * np.float32(1.01))
    return out


def _sa_ra_jump():
    row = jax.lax.broadcasted_iota(jnp.int32, (2, 1), 0)
    return jnp.where(row == 0, jnp.float32(6.0), jnp.float32(2.0))


def _snn_body(stim_ref, e5_ref, e10_ref, a_sa_ref,
              sa_rf_ref, ra_rf_ref,
              sa_cn_in_ref, sa_cn_pn_ref, sa_io_ref, ra_cn_in_ref,
              ra_cn_pn_ref, ra_io_ref, cn_in_sa_ref, cn_pn_sa_ref,
              cn_in_ra_ref, cn_pn_ra_ref, cn_io_ref,
              sa_cn_sd_ref, sa_io_sd_ref, ra_cn_sd_ref, ra_io_sd_ref,
              cn_sa_sd_ref, cn_ra_sd_ref, cn_io_sd_ref,
              out_ref,
              sa0_f, ra0_f, p12_f, p22_f, spk_cn0, spk_cn1,
              hu_sa0, hu_ra0, hu_12, hu_22,
              cur_a, cur_b, cur_c, cur_d):
    f32 = jnp.float32

    def shift_matmul(hist, w_refs, sd_ref, post_each):
        sd = sd_ref[...]
        nw = len(w_refs)
        w_cat = (w_refs[0][...] if nw == 1 else
                 jnp.concatenate([w[...] for w in w_refs], axis=0))
        parts = [jnp.zeros((TS, nw * post_each), f32) for _ in range(2)]
        for d in range(1, MAX_DELAY + 1):
            mask = (sd == d).astype(f32)
            sl = hist[PAD - d:PAD - d + TS, :]
            wd = w_cat * (jnp.concatenate([mask] * nw, axis=0)
                          if nw > 1 else mask)
            parts[d % 2] = parts[d % 2] + jax.lax.dot_general(
                sl, wd, _DN_RR, precision=_HI, preferred_element_type=f32)
        acc = parts[0] + parts[1]
        return [acc[:, i * post_each:(i + 1) * post_each] for i in range(nw)]

    def scan(layers):
        inits = []
        for (_, _, m, width, _, _) in layers:
            inits.append(jnp.full((m, width), -65.0, f32))
            inits.append(jnp.full((m, width), -13.0, f32))

        def body(k, carry):
            r8 = k * 8
            a_blk = a_sa_ref[pl.ds(r8, 8), :]
            out = []
            for i, (cur_ref, spk_ref, m, width, d_jump, a_mode) in enumerate(layers):
                v, u = carry[2 * i], carry[2 * i + 1]
                blk = cur_ref[pl.ds(8 * m * k, 8 * m), 0:width]
                rows = []
                for j in range(8):
                    I = blk[m * j:m * (j + 1), :]
                    v = v + 0.5 * (0.04 * v * v + 5.0 * v + 140.0 - u + I)
                    v = v + 0.5 * (0.04 * v * v + 5.0 * v + 140.0 - u + I)
                    a = a_blk[j:j + 1, :] if a_mode == 'sa0' else a_mode
                    u = u + a * (0.2 * v - u)
                    spk = (v >= V_THRES).astype(f32)
                    v = jnp.where(spk > 0, -65.0, v)
                    u = jnp.where(spk > 0, u + d_jump, u)
                    rows.append(spk)
                spk_ref[pl.ds(m * (PAD + r8), 8 * m), :] = jnp.concatenate(
                    rows, axis=0)
                out.append(v)
                out.append(u)
            return tuple(out)

        jax.lax.fori_loop(0, NBLK, body, tuple(inits))

    for ref, m, n in ((sa0_f, 4, 128), (ra0_f, 4, 128), (p12_f, 2, 128),
                      (p22_f, 2, 128), (spk_cn0, 1, N_CN), (spk_cn1, 1, N_CN)):
        ref[0:PAD * m, :] = jnp.zeros((PAD * m, n), jnp.float32)

    stim_t = jnp.transpose(stim_ref[...], (1, 0))
    stim_sa = stim_t[1:T, :]
    stim_ra = jnp.abs(stim_t[1:T, :] - stim_t[0:T - 1, :]) * 5.0
    e5 = e5_ref[...]
    e10 = e10_ref[...]
    psp_sa = jnp.dot(e5, stim_sa, precision=_HI, preferred_element_type=f32)
    psp_ra = jnp.dot(e5, stim_ra, precision=_HI, preferred_element_type=f32)
    i_sa0 = jax.lax.dot_general(psp_sa, sa_rf_ref[...], _DN_RR, precision=_HI,
                                preferred_element_type=f32)
    i_ra0 = jax.lax.dot_general(psp_ra, ra_rf_ref[...], _DN_RR, precision=_HI,
                                preferred_element_type=f32)
    cur_a[0:4 * TS, :] = jnp.reshape(i_sa0, (4 * TS, 128))
    cur_b[0:4 * TS, :] = jnp.reshape(i_ra0, (4 * TS, 128))
    scan([(cur_a, sa0_f, 4, 128, 8.0, 'sa0'),
          (cur_b, ra0_f, 4, 128, 2.0, 0.02)])
    hu_sa0[...] = jnp.reshape(sa0_f[...], (HROWS, N_SA))
    hu_ra0[...] = jnp.reshape(ra0_f[...], (HROWS, N_RA))

    c_sa_in, c_sa_pn = shift_matmul(hu_sa0, (sa_cn_in_ref, sa_cn_pn_ref),
                                    sa_cn_sd_ref, N_IN)
    c_ra_in, c_ra_pn = shift_matmul(hu_ra0, (ra_cn_in_ref, ra_cn_pn_ref),
                                    ra_cn_sd_ref, N_IN)
    i_sa1 = jnp.dot(e5, c_sa_in, precision=_HI, preferred_element_type=f32)
    i_ra1 = jnp.dot(e5, c_ra_in, precision=_HI, preferred_element_type=f32)
    pn1 = jnp.dot(e5, c_sa_pn, precision=_HI, preferred_element_type=f32)
    rpn1 = jnp.dot(e5, c_ra_pn, precision=_HI, preferred_element_type=f32)
    cur_c[0:2 * TS, :] = jnp.reshape(
        jnp.concatenate([i_sa1, i_ra1], axis=1), (2 * TS, 128))
    scan([(cur_c, p12_f, 2, 128, _sa_ra_jump(), 0.1)])
    hu_12[...] = jnp.reshape(p12_f[...], (HROWS, 2 * N_IN))

    (c_saio,) = shift_matmul(hu_12[:, 0:N_IN], (sa_io_ref,),
                             sa_io_sd_ref, N_PN)
    (c_raio,) = shift_matmul(hu_12[:, N_IN:2 * N_IN], (ra_io_ref,),
                             ra_io_sd_ref, N_PN)
    i_sa2 = pn1 - jnp.dot(e10, c_saio, precision=_HI,
                          preferred_element_type=f32)
    i_ra2 = rpn1 - jnp.dot(e10, c_raio, precision=_HI,
                           preferred_element_type=f32)
    cur_d[0:2 * TS, :] = jnp.reshape(
        jnp.concatenate([i_sa2, i_ra2], axis=1), (2 * TS, 128))
    scan([(cur_d, p22_f, 2, 128, _sa_ra_jump(), 0.1)])
    hu_22[...] = jnp.reshape(p22_f[...], (HROWS, 2 * N_PN))

    c_cnsa_in, c_cnsa_pn = shift_matmul(hu_22[:, 0:N_PN],
                                        (cn_in_sa_ref, cn_pn_sa_ref),
                                        cn_sa_sd_ref, N_CN)
    c_cnra_in, c_cnra_pn = shift_matmul(hu_22[:, N_PN:2 * N_PN],
                                        (cn_in_ra_ref, cn_pn_ra_ref),
                                        cn_ra_sd_ref, N_CN)
    cur_a[0:TS, 0:N_CN] = jnp.dot(e5, c_cnsa_in + c_cnra_in, precision=_HI,
                                  preferred_element_type=f32)
    scan([(cur_a, spk_cn0, 1, N_CN, 8.0, 0.02)])

    (c_cnio,) = shift_matmul(spk_cn0, (cn_io_ref,), cn_io_sd_ref, N_CN)
    cur_b[0:TS, 0:N_CN] = (
        2.0 * jnp.dot(e5, c_cnsa_pn + c_cnra_pn, precision=_HI,
                      preferred_element_type=f32)
        - jnp.dot(e10, c_cnio, precision=_HI, preferred_element_type=f32))
    scan([(cur_b, spk_cn1, 1, N_CN, 8.0, 0.02)])

    for ref, off, n in ((hu_sa0, 0, N_SA),
                        (hu_12, 512, N_IN),
                        (hu_22, 640, N_PN),
                        (hu_ra0, 768, N_RA),
                        (spk_cn0, 1536, N_CN), (spk_cn1, 1600, N_CN)):
        out_ref[off:off + n, :] = jnp.transpose(ref[PAD:PAD + TS, 0:n], (1, 0))
    out_ref[1280:1280 + N_IN, :] = jnp.transpose(
        hu_12[PAD:PAD + TS, N_IN:2 * N_IN], (1, 0))
    out_ref[1408:1408 + N_PN, :] = jnp.transpose(
        hu_22[PAD:PAD + TS, N_PN:2 * N_PN], (1, 0))


def kernel(stim, sa_rf, sa_cn_in_rf, sa_cn_pn_rf, sa_intopn_rf,
           ra_rf, ra_cn_in_rf, ra_cn_pn_rf, ra_intopn_rf,
           cn_in_sa_rf, cn_pn_sa_rf, cn_in_ra_rf, cn_pn_ra_rf, cn_intopn_rf,
           sa_cn_SD, sa_intopn_DN, ra_cn_SD, ra_intopn_DN,
           cn_sa_SD, cn_ra_SD, cn_intopn_DN):
    f32 = jnp.float32

    e5 = jnp.asarray(_decay_filter(5.0))
    e10 = jnp.asarray(_decay_filter(10.0))
    a_sa = jnp.asarray(_a_schedule())

    scratch = [
        pltpu.VMEM((4 * HROWS, 128), f32),
        pltpu.VMEM((4 * HROWS, 128), f32),
        pltpu.VMEM((2 * HROWS, 128), f32),
        pltpu.VMEM((2 * HROWS, 128), f32),
        pltpu.VMEM((HROWS, N_CN), f32),
        pltpu.VMEM((HROWS, N_CN), f32),
        pltpu.VMEM((HROWS, N_SA), f32),
        pltpu.VMEM((HROWS, N_RA), f32),
        pltpu.VMEM((HROWS, 2 * N_IN), f32),
        pltpu.VMEM((HROWS, 2 * N_PN), f32),
        pltpu.VMEM((4 * CROWS, 128), f32),
        pltpu.VMEM((4 * CROWS, 128), f32),
        pltpu.VMEM((2 * CROWS, 128), f32),
        pltpu.VMEM((2 * CROWS, 128), f32),
    ]

    out = pl.pallas_call(
        _snn_body,
        out_shape=jax.ShapeDtypeStruct((1664, TS), f32),
        scratch_shapes=scratch,
    )(stim[0], e5, e10, a_sa,
      sa_rf, ra_rf,
      sa_cn_in_rf, sa_cn_pn_rf, sa_intopn_rf, ra_cn_in_rf,
      ra_cn_pn_rf, ra_intopn_rf, cn_in_sa_rf, cn_pn_sa_rf,
      cn_in_ra_rf, cn_pn_ra_rf, cn_intopn_rf,
      sa_cn_SD.astype(jnp.int32), sa_intopn_DN.astype(jnp.int32),
      ra_cn_SD.astype(jnp.int32), ra_intopn_DN.astype(jnp.int32),
      cn_sa_SD.astype(jnp.int32), cn_ra_SD.astype(jnp.int32),
      cn_intopn_DN.astype(jnp.int32))

    return out


# speedup vs baseline: 1.3874x; 1.3874x over previous
import jax
import jax.numpy as jnp
import numpy as np
from jax.experimental import pallas as pl
from jax.experimental.pallas import tpu as pltpu

N_SA = 512; N_RA = 512; N_IN = 128; N_PN = 128; N_CN = 64
T = 120; MAX_DELAY = 10; BUF = MAX_DELAY + 1
TS = T - 1
PAD = 16
NBLK = (TS + 7) // 8
HROWS = PAD + 8 * NBLK + 8
CROWS = 8 * NBLK
V_THRES = 30.0

_HI = jax.lax.Precision.HIGHEST
_DN_RR = (((1,), (1,)), ((), ()))


def _decay_filter(tau: float) -> np.ndarray:
    e = np.float32(np.exp(np.float32(-1.0) / np.float32(tau)))
    k = np.arange(TS)
    pw = np.power(np.float64(e), k).astype(np.float32)
    L = np.zeros((TS, TS), np.float32)
    for r in range(TS):
        L[r, : r + 1] = pw[: r + 1][::-1]
    return L


def _a_schedule() -> np.ndarray:
    a = np.float32(0.02)
    out = np.zeros((CROWS, 1), np.float32)
    for r in range(TS):
        out[r, 0] = a
        a = np.float32(a * np.float32(1.01))
    return out


def _sa_ra_jump():
    row = jax.lax.broadcasted_iota(jnp.int32, (2, 1), 0)
    return jnp.where(row == 0, jnp.float32(6.0), jnp.float32(2.0))


def _snn_body(stim_ref, e5_ref, e10_ref, a_sa_ref,
              sa_rf_ref, ra_rf_ref,
              sa_cn_in_ref, sa_cn_pn_ref, sa_io_ref, ra_cn_in_ref,
              ra_cn_pn_ref, ra_io_ref, cn_in_sa_ref, cn_pn_sa_ref,
              cn_in_ra_ref, cn_pn_ra_ref, cn_io_ref,
              sa_cn_sd_ref, sa_io_sd_ref, ra_cn_sd_ref, ra_io_sd_ref,
              cn_sa_sd_ref, cn_ra_sd_ref, cn_io_sd_ref,
              out_ref,
              sa0_f, ra0_f, p12_f, p22_f, spk_cn0, spk_cn1,
              hu_sa0, hu_ra0, hu_12, hu_22,
              cur_a, cur_b, cur_c, cur_d):
    f32 = jnp.float32

    def shift_matmul(hist, w_refs, sd_ref, post_each):
        sd = sd_ref[...]
        bf16 = jnp.bfloat16
        w_splits = []
        for w_ref in w_refs:
            w = w_ref[...]
            h1 = w.astype(bf16)
            r1 = w - h1.astype(f32)
            h2 = r1.astype(bf16)
            h3 = (r1 - h2.astype(f32)).astype(bf16)
            w_splits.append((h1, h2, h3))
        hist16 = hist[...].astype(bf16)
        accs = [jnp.zeros((TS, post_each), f32) for _ in w_refs]
        for d in range(1, MAX_DELAY + 1):
            mask = (sd == d).astype(bf16)
            sl = hist16[PAD - d:PAD - d + TS, :]
            for i, parts in enumerate(w_splits):
                for h in parts:
                    accs[i] = accs[i] + jax.lax.dot_general(
                        sl, h * mask, _DN_RR,
                        preferred_element_type=f32)
        return accs

    def scan(layers):
        inits = []
        for (_, _, m, width, _, _) in layers:
            inits.append(jnp.full((m, width), -65.0, f32))
            inits.append(jnp.full((m, width), -13.0, f32))

        def body(k, carry):
            r8 = k * 8
            a_blk = a_sa_ref[pl.ds(r8, 8), :]
            out = []
            for i, (cur_ref, spk_ref, m, width, d_jump, a_mode) in enumerate(layers):
                v, u = carry[2 * i], carry[2 * i + 1]
                blk = cur_ref[pl.ds(8 * m * k, 8 * m), 0:width]
                rows = []
                for j in range(8):
                    I = blk[m * j:m * (j + 1), :]
                    v = v + 0.5 * (0.04 * v * v + 5.0 * v + 140.0 - u + I)
                    v = v + 0.5 * (0.04 * v * v + 5.0 * v + 140.0 - u + I)
                    a = a_blk[j:j + 1, :] if a_mode == 'sa0' else a_mode
                    u = u + a * (0.2 * v - u)
                    spk = (v >= V_THRES).astype(f32)
                    v = jnp.where(spk > 0, -65.0, v)
                    u = jnp.where(spk > 0, u + d_jump, u)
                    rows.append(spk)
                spk_ref[pl.ds(m * (PAD + r8), 8 * m), :] = jnp.concatenate(
                    rows, axis=0)
                out.append(v)
                out.append(u)
            return tuple(out)

        jax.lax.fori_loop(0, NBLK, body, tuple(inits))

    for ref, m, n in ((sa0_f, 4, 128), (ra0_f, 4, 128), (p12_f, 2, 128),
                      (p22_f, 2, 128), (spk_cn0, 1, N_CN), (spk_cn1, 1, N_CN)):
        ref[0:PAD * m, :] = jnp.zeros((PAD * m, n), jnp.float32)

    stim_t = jnp.transpose(stim_ref[...], (1, 0))
    stim_sa = stim_t[1:T, :]
    stim_ra = jnp.abs(stim_t[1:T, :] - stim_t[0:T - 1, :]) * 5.0
    e5 = e5_ref[...]
    e10 = e10_ref[...]
    psp_sa = jnp.dot(e5, stim_sa, precision=_HI, preferred_element_type=f32)
    psp_ra = jnp.dot(e5, stim_ra, precision=_HI, preferred_element_type=f32)
    i_sa0 = jax.lax.dot_general(psp_sa, sa_rf_ref[...], _DN_RR, precision=_HI,
                                preferred_element_type=f32)
    i_ra0 = jax.lax.dot_general(psp_ra, ra_rf_ref[...], _DN_RR, precision=_HI,
                                preferred_element_type=f32)
    cur_a[0:4 * TS, :] = jnp.reshape(i_sa0, (4 * TS, 128))
    cur_b[0:4 * TS, :] = jnp.reshape(i_ra0, (4 * TS, 128))
    scan([(cur_a, sa0_f, 4, 128, 8.0, 'sa0'),
          (cur_b, ra0_f, 4, 128, 2.0, 0.02)])
    hu_sa0[...] = jnp.reshape(sa0_f[...], (HROWS, N_SA))
    hu_ra0[...] = jnp.reshape(ra0_f[...], (HROWS, N_RA))

    c_sa_in, c_sa_pn = shift_matmul(hu_sa0, (sa_cn_in_ref, sa_cn_pn_ref),
                                    sa_cn_sd_ref, N_IN)
    c_ra_in, c_ra_pn = shift_matmul(hu_ra0, (ra_cn_in_ref, ra_cn_pn_ref),
                                    ra_cn_sd_ref, N_IN)
    i_sa1 = jnp.dot(e5, c_sa_in, precision=_HI, preferred_element_type=f32)
    i_ra1 = jnp.dot(e5, c_ra_in, precision=_HI, preferred_element_type=f32)
    pn1 = jnp.dot(e5, c_sa_pn, precision=_HI, preferred_element_type=f32)
    rpn1 = jnp.dot(e5, c_ra_pn, precision=_HI, preferred_element_type=f32)
    cur_c[0:2 * TS, :] = jnp.reshape(
        jnp.concatenate([i_sa1, i_ra1], axis=1), (2 * TS, 128))
    scan([(cur_c, p12_f, 2, 128, _sa_ra_jump(), 0.1)])
    hu_12[...] = jnp.reshape(p12_f[...], (HROWS, 2 * N_IN))

    (c_saio,) = shift_matmul(hu_12[:, 0:N_IN], (sa_io_ref,),
                             sa_io_sd_ref, N_PN)
    (c_raio,) = shift_matmul(hu_12[:, N_IN:2 * N_IN], (ra_io_ref,),
                             ra_io_sd_ref, N_PN)
    i_sa2 = pn1 - jnp.dot(e10, c_saio, precision=_HI,
                          preferred_element_type=f32)
    i_ra2 = rpn1 - jnp.dot(e10, c_raio, precision=_HI,
                           preferred_element_type=f32)
    cur_d[0:2 * TS, :] = jnp.reshape(
        jnp.concatenate([i_sa2, i_ra2], axis=1), (2 * TS, 128))
    scan([(cur_d, p22_f, 2, 128, _sa_ra_jump(), 0.1)])
    hu_22[...] = jnp.reshape(p22_f[...], (HROWS, 2 * N_PN))

    c_cnsa_in, c_cnsa_pn = shift_matmul(hu_22[:, 0:N_PN],
                                        (cn_in_sa_ref, cn_pn_sa_ref),
                                        cn_sa_sd_ref, N_CN)
    c_cnra_in, c_cnra_pn = shift_matmul(hu_22[:, N_PN:2 * N_PN],
                                        (cn_in_ra_ref, cn_pn_ra_ref),
                                        cn_ra_sd_ref, N_CN)
    cur_a[0:TS, 0:N_CN] = jnp.dot(e5, c_cnsa_in + c_cnra_in, precision=_HI,
                                  preferred_element_type=f32)
    scan([(cur_a, spk_cn0, 1, N_CN, 8.0, 0.02)])

    (c_cnio,) = shift_matmul(spk_cn0, (cn_io_ref,), cn_io_sd_ref, N_CN)
    cur_b[0:TS, 0:N_CN] = (
        2.0 * jnp.dot(e5, c_cnsa_pn + c_cnra_pn, precision=_HI,
                      preferred_element_type=f32)
        - jnp.dot(e10, c_cnio, precision=_HI, preferred_element_type=f32))
    scan([(cur_b, spk_cn1, 1, N_CN, 8.0, 0.02)])

    for ref, off, n in ((hu_sa0, 0, N_SA),
                        (hu_12, 512, N_IN),
                        (hu_22, 640, N_PN),
                        (hu_ra0, 768, N_RA),
                        (spk_cn0, 1536, N_CN), (spk_cn1, 1600, N_CN)):
        out_ref[off:off + n, :] = jnp.transpose(ref[PAD:PAD + TS, 0:n], (1, 0))
    out_ref[1280:1280 + N_IN, :] = jnp.transpose(
        hu_12[PAD:PAD + TS, N_IN:2 * N_IN], (1, 0))
    out_ref[1408:1408 + N_PN, :] = jnp.transpose(
        hu_22[PAD:PAD + TS, N_PN:2 * N_PN], (1, 0))


def kernel(stim, sa_rf, sa_cn_in_rf, sa_cn_pn_rf, sa_intopn_rf,
           ra_rf, ra_cn_in_rf, ra_cn_pn_rf, ra_intopn_rf,
           cn_in_sa_rf, cn_pn_sa_rf, cn_in_ra_rf, cn_pn_ra_rf, cn_intopn_rf,
           sa_cn_SD, sa_intopn_DN, ra_cn_SD, ra_intopn_DN,
           cn_sa_SD, cn_ra_SD, cn_intopn_DN):
    f32 = jnp.float32

    e5 = jnp.asarray(_decay_filter(5.0))
    e10 = jnp.asarray(_decay_filter(10.0))
    a_sa = jnp.asarray(_a_schedule())

    scratch = [
        pltpu.VMEM((4 * HROWS, 128), f32),
        pltpu.VMEM((4 * HROWS, 128), f32),
        pltpu.VMEM((2 * HROWS, 128), f32),
        pltpu.VMEM((2 * HROWS, 128), f32),
        pltpu.VMEM((HROWS, N_CN), f32),
        pltpu.VMEM((HROWS, N_CN), f32),
        pltpu.VMEM((HROWS, N_SA), f32),
        pltpu.VMEM((HROWS, N_RA), f32),
        pltpu.VMEM((HROWS, 2 * N_IN), f32),
        pltpu.VMEM((HROWS, 2 * N_PN), f32),
        pltpu.VMEM((4 * CROWS, 128), f32),
        pltpu.VMEM((4 * CROWS, 128), f32),
        pltpu.VMEM((2 * CROWS, 128), f32),
        pltpu.VMEM((2 * CROWS, 128), f32),
    ]

    out = pl.pallas_call(
        _snn_body,
        out_shape=jax.ShapeDtypeStruct((1664, TS), f32),
        scratch_shapes=scratch,
    )(stim[0], e5, e10, a_sa,
      sa_rf, ra_rf,
      sa_cn_in_rf, sa_cn_pn_rf, sa_intopn_rf, ra_cn_in_rf,
      ra_cn_pn_rf, ra_intopn_rf, cn_in_sa_rf, cn_pn_sa_rf,
      cn_in_ra_rf, cn_pn_ra_rf, cn_intopn_rf,
      sa_cn_SD.astype(jnp.int32), sa_intopn_DN.astype(jnp.int32),
      ra_cn_SD.astype(jnp.int32), ra_intopn_DN.astype(jnp.int32),
      cn_sa_SD.astype(jnp.int32), cn_ra_SD.astype(jnp.int32),
      cn_intopn_DN.astype(jnp.int32))

    return out
